# baseline (speedup 1.0000x reference)
import jax
import jax.numpy as jnp
from jax.experimental import pallas as pl
from jax.experimental.pallas import tpu as pltpu

_LANES = 128
_BRY = 2048


def _clip_body(y_ref, ob_ref, o_ref):
    bry = y_ref.shape[0]
    y = y_ref[...]
    a = ob_ref[pl.ds(0, bry, 2), :]
    b = ob_ref[pl.ds(1, bry, 2), :]

    lane = jax.lax.broadcasted_iota(jnp.int32, (bry, _LANES), 1)
    idx = 2 * (lane % 64) + (lane // 64)
    pa = jnp.take_along_axis(a, idx, axis=1)
    pb = jnp.take_along_axis(b, idx, axis=1)
    ra = pltpu.roll(pa, 64, axis=1)
    rb = pltpu.roll(pb, 64, axis=1)
    lo_half = lane < 64
    lower = jnp.where(lo_half, pa, rb)
    upper = jnp.where(lo_half, ra, pb)
    o_ref[...] = jnp.clip(y, lower, upper)


def kernel(Y_hat, obs):
    B = Y_hat.shape[0]
    nry = B // _LANES
    yv = Y_hat.reshape(nry, _LANES)
    obv = obs.reshape(2 * nry, _LANES)
    grid = (nry // _BRY,)
    out = pl.pallas_call(
        _clip_body,
        grid=grid,
        in_specs=[
            pl.BlockSpec((_BRY, _LANES), lambda i: (i, 0)),
            pl.BlockSpec((2 * _BRY, _LANES), lambda i: (i, 0)),
        ],
        out_specs=pl.BlockSpec((_BRY, _LANES), lambda i: (i, 0)),
        out_shape=jax.ShapeDtypeStruct((nry, _LANES), Y_hat.dtype),
        compiler_params=pltpu.CompilerParams(
            dimension_semantics=("parallel",),
        ),
    )(yv, obv)
    return out.reshape(B, 1)


# speedup vs baseline: 250.1223x; 250.1223x over previous
import jax
import jax.numpy as jnp
from jax.experimental import pallas as pl
from jax.experimental.pallas import tpu as pltpu

_LANES = 128
_BRY = 2048


def _clip_body(y_ref, ob_ref, o_ref):
    bry = y_ref.shape[0]
    y = y_ref[...]
    lower = ob_ref[pl.ds(0, bry, 2), :]
    upper = ob_ref[pl.ds(1, bry, 2), :]
    o_ref[...] = jnp.clip(y, lower, upper)


def kernel(Y_hat, obs):
    B = Y_hat.shape[0]
    nry = B // _LANES
    yv = Y_hat.reshape(nry, _LANES)
    obv = obs.reshape(nry, _LANES, 2).transpose(0, 2, 1).reshape(2 * nry, _LANES)
    grid = (nry // _BRY,)
    out = pl.pallas_call(
        _clip_body,
        grid=grid,
        in_specs=[
            pl.BlockSpec((_BRY, _LANES), lambda i: (i, 0)),
            pl.BlockSpec((2 * _BRY, _LANES), lambda i: (i, 0)),
        ],
        out_specs=pl.BlockSpec((_BRY, _LANES), lambda i: (i, 0)),
        out_shape=jax.ShapeDtypeStruct((nry, _LANES), Y_hat.dtype),
        compiler_params=pltpu.CompilerParams(
            dimension_semantics=("parallel",),
        ),
    )(yv, obv)
    return out.reshape(B, 1)


# speedup vs baseline: 268.1246x; 1.0720x over previous
import jax
import jax.numpy as jnp
from jax.experimental import pallas as pl
from jax.experimental.pallas import tpu as pltpu

_LANES = 128
_BRY = 4096


def _clip_body(y_ref, ob_ref, o_ref):
    bry = y_ref.shape[0]
    y = y_ref[...]
    lower = ob_ref[pl.ds(0, bry, 2), :]
    upper = ob_ref[pl.ds(1, bry, 2), :]
    o_ref[...] = jnp.clip(y, lower, upper)


def kernel(Y_hat, obs):
    B = Y_hat.shape[0]
    nry = B // _LANES
    yv = Y_hat.reshape(nry, _LANES)
    obv = obs.reshape(nry, _LANES, 2).transpose(0, 2, 1).reshape(2 * nry, _LANES)
    grid = (nry // _BRY,)
    out = pl.pallas_call(
        _clip_body,
        grid=grid,
        in_specs=[
            pl.BlockSpec((_BRY, _LANES), lambda i: (i, 0)),
            pl.BlockSpec((2 * _BRY, _LANES), lambda i: (i, 0)),
        ],
        out_specs=pl.BlockSpec((_BRY, _LANES), lambda i: (i, 0)),
        out_shape=jax.ShapeDtypeStruct((nry, _LANES), Y_hat.dtype),
        compiler_params=pltpu.CompilerParams(
            dimension_semantics=("parallel",),
        ),
    )(yv, obv)
    return out.reshape(B, 1)
